# baseline (speedup 1.0000x reference)
import jax
import jax.numpy as jnp
from jax.experimental import pallas as pl
from jax.experimental.pallas import tpu as pltpu
from jax.experimental.pallas import tpu_sc as plsc

_BATCH = 4096
_SEQ = 200
_EMB = 64
_B = _BATCH * _SEQ
_NW = 32
_N_PER_W = _B // _NW
_W = 128
_NWIN = _N_PER_W // _W
_NBUF = 4


_VOCAB = 1000000
_PAD_ROWS = 2000


def _pad_block(x_ref, o_ref):
    o_ref[:, : _EMB] = x_ref[...]


def _pad_table_tc(table):
    return pl.pallas_call(
        _pad_block,
        grid=(_VOCAB // _PAD_ROWS,),
        in_specs=[pl.BlockSpec((_PAD_ROWS, _EMB), lambda i: (i, 0))],
        out_specs=pl.BlockSpec((_PAD_ROWS, 128), lambda i: (i, 0)),
        out_shape=jax.ShapeDtypeStruct((_VOCAB, 128), table.dtype),
    )(table)


def kernel(token_ids, table):
    idx = token_ids.reshape(_NW, _NWIN, _W).astype(jnp.int32)
    tab128 = _pad_table_tc(table)
    mesh = plsc.VectorSubcoreMesh(core_axis_name="core", subcore_axis_name="subcore")

    @pl.kernel(
        out_type=jax.ShapeDtypeStruct((_B, 128), table.dtype),
        mesh=mesh,
        compiler_params=pltpu.CompilerParams(use_tc_tiling_on_sc=True),
        scratch_types=[
            pltpu.VMEM((_NWIN, _W), jnp.int32),
            pltpu.VMEM((_NBUF, _W, 128), jnp.float32),
            pltpu.SemaphoreType.DMA((_NBUF,)),
            pltpu.SemaphoreType.DMA((_NBUF,)),
            pltpu.SemaphoreType.DMA,
        ],
    )
    def k(tab_hbm, i_hbm, o_hbm, idx_v, bufs, gsem, osem, isem):
        wid = jax.lax.axis_index("subcore") * 2 + jax.lax.axis_index("core")
        base = wid * _N_PER_W

        pltpu.async_copy(i_hbm.at[wid], idx_v, isem).wait()

        def start_gather(win, b):
            pltpu.make_async_copy(
                tab_hbm.at[idx_v.at[win]], bufs.at[b], gsem.at[b]
            ).start()

        def drain_slot(win, b):
            pltpu.make_async_copy(
                tab_hbm.at[idx_v.at[win]], bufs.at[b], gsem.at[b]
            ).wait()
            cp = pltpu.make_async_copy(
                bufs.at[b], o_hbm.at[pl.ds(base + win * _W, _W)], osem.at[b]
            )
            cp.start()
            cp.wait()

        for b in range(_NBUF):
            start_gather(b, b)

        @pl.loop(_NBUF, _NWIN, step=_NBUF)
        def _(g0):
            for b in range(_NBUF):
                drain_slot(g0 - _NBUF + b, b)
                start_gather(g0 + b, b)

        for b in range(_NBUF):
            drain_slot(_NWIN - _NBUF + b, b)

    out128 = k(tab128, idx)
    return out128[:, :_EMB].reshape(_BATCH, _SEQ, _EMB)


# speedup vs baseline: 1.3040x; 1.3040x over previous
import jax
import jax.numpy as jnp
from jax.experimental import pallas as pl
from jax.experimental.pallas import tpu as pltpu
from jax.experimental.pallas import tpu_sc as plsc

_VOCAB = 1000000
_BATCH = 4096
_SEQ = 200
_EMB = 64
_B = _BATCH * _SEQ
_NW = 32
_N_PER_W = _B // _NW
_W = 64
_NWIN = _N_PER_W // _W
_G = 4
_NR = _NWIN // _G


def kernel(token_ids, table):
    idx = token_ids.reshape(_NW, _NWIN, _W).astype(jnp.int32)
    tab128 = jnp.pad(table, ((0, 0), (0, 128 - _EMB)))
    mesh = plsc.VectorSubcoreMesh(core_axis_name="core", subcore_axis_name="subcore")

    @pl.kernel(
        out_type=jax.ShapeDtypeStruct((_B, 128), table.dtype),
        mesh=mesh,
        compiler_params=pltpu.CompilerParams(use_tc_tiling_on_sc=True),
        scratch_types=[
            pltpu.VMEM((_NWIN, _W), jnp.int32),
            pltpu.VMEM((2 * _G, _W, 128), jnp.float32),
            pltpu.SemaphoreType.DMA((2 * _G,)),
            pltpu.SemaphoreType.DMA((2 * _G,)),
            pltpu.SemaphoreType.DMA,
        ],
    )
    def k(tab_hbm, i_hbm, o_hbm, idx_v, bufs, gsem, osem, isem):
        wid = jax.lax.axis_index("subcore") * 2 + jax.lax.axis_index("core")
        base = wid * _N_PER_W

        pltpu.async_copy(i_hbm.at[wid], idx_v, isem).wait()

        def start_gather(win, s):
            pltpu.make_async_copy(
                tab_hbm.at[idx_v.at[win]], bufs.at[s], gsem.at[s]
            ).start()

        def wait_gather_start_out(win, s):
            pltpu.make_async_copy(
                tab_hbm.at[idx_v.at[win]], bufs.at[s], gsem.at[s]
            ).wait()
            pltpu.make_async_copy(
                bufs.at[s], o_hbm.at[pl.ds(base + win * _W, _W)], osem.at[s]
            ).start()

        def wait_out(win, s):
            pltpu.make_async_copy(
                bufs.at[s], o_hbm.at[pl.ds(base + win * _W, _W)], osem.at[s]
            ).wait()

        for l in range(_G):
            start_gather(l, l)
        for l in range(_G):
            wait_gather_start_out(l, l)
        for l in range(_G):
            start_gather(_G + l, _G + l)

        @pl.loop(2, _NR)
        def _(r):
            grp = (r % 2) * _G
            pgrp = ((r - 1) % 2) * _G
            for l in range(_G):
                wait_gather_start_out((r - 1) * _G + l, pgrp + l)
            for l in range(_G):
                wait_out((r - 2) * _G + l, grp + l)
                start_gather(r * _G + l, grp + l)

        lgrp = ((_NR - 1) % 2) * _G
        for l in range(_G):
            wait_gather_start_out((_NR - 1) * _G + l, lgrp + l)
        for l in range(_G):
            wait_out((_NR - 2) * _G + l, (lgrp ^ _G) + l)
            wait_out((_NR - 1) * _G + l, lgrp + l)

    out128 = k(tab128, idx)
    return out128[:, :_EMB].reshape(_BATCH, _SEQ, _EMB)


# speedup vs baseline: 1.3088x; 1.0037x over previous
import jax
import jax.numpy as jnp
from jax.experimental import pallas as pl
from jax.experimental.pallas import tpu as pltpu
from jax.experimental.pallas import tpu_sc as plsc

_VOCAB = 1000000
_BATCH = 4096
_SEQ = 200
_EMB = 64
_B = _BATCH * _SEQ
_NW = 32
_N_PER_W = _B // _NW
_W = 128
_NWIN = _N_PER_W // _W
_G = 2
_NR = _NWIN // _G


def kernel(token_ids, table):
    idx = token_ids.reshape(_NW, _NWIN, _W).astype(jnp.int32)
    tab128 = jnp.pad(table, ((0, 0), (0, 128 - _EMB)))
    mesh = plsc.VectorSubcoreMesh(core_axis_name="core", subcore_axis_name="subcore")

    @pl.kernel(
        out_type=jax.ShapeDtypeStruct((_B, 128), table.dtype),
        mesh=mesh,
        compiler_params=pltpu.CompilerParams(use_tc_tiling_on_sc=True),
        scratch_types=[
            pltpu.VMEM((_NWIN, _W), jnp.int32),
            pltpu.VMEM((3 * _G, _W, 128), jnp.float32),
            pltpu.SemaphoreType.DMA((3 * _G,)),
            pltpu.SemaphoreType.DMA((3 * _G,)),
            pltpu.SemaphoreType.DMA,
        ],
    )
    def k(tab_hbm, i_hbm, o_hbm, idx_v, bufs, gsem, osem, isem):
        wid = jax.lax.axis_index("subcore") * 2 + jax.lax.axis_index("core")
        base = wid * _N_PER_W

        pltpu.async_copy(i_hbm.at[wid], idx_v, isem).wait()

        def start_gather(win, s):
            pltpu.make_async_copy(
                tab_hbm.at[idx_v.at[win]], bufs.at[s], gsem.at[s]
            ).start()

        def wait_gather_start_out(win, s):
            pltpu.make_async_copy(
                tab_hbm.at[idx_v.at[win]], bufs.at[s], gsem.at[s]
            ).wait()
            pltpu.make_async_copy(
                bufs.at[s], o_hbm.at[pl.ds(base + win * _W, _W)], osem.at[s]
            ).start()

        def wait_out(win, s):
            pltpu.make_async_copy(
                bufs.at[s], o_hbm.at[pl.ds(base + win * _W, _W)], osem.at[s]
            ).wait()

        for l in range(_G):
            start_gather(l, l)
        for r0 in (1, 2):
            for l in range(_G):
                start_gather(r0 * _G + l, (r0 % 3) * _G + l)
            for l in range(_G):
                wait_gather_start_out((r0 - 1) * _G + l, ((r0 - 1) % 3) * _G + l)

        @pl.loop(3, _NR)
        def _(r):
            grp = (r % 3) * _G
            pgrp = ((r - 1) % 3) * _G
            for l in range(_G):
                wait_out((r - 3) * _G + l, grp + l)
                start_gather(r * _G + l, grp + l)
            for l in range(_G):
                wait_gather_start_out((r - 1) * _G + l, pgrp + l)

        last = _NR - 1
        for l in range(_G):
            wait_gather_start_out(last * _G + l, (last % 3) * _G + l)
        for r0 in (last - 2, last - 1, last):
            for l in range(_G):
                wait_out(r0 * _G + l, (r0 % 3) * _G + l)

    out128 = k(tab128, idx)
    return out128[:, :_EMB].reshape(_BATCH, _SEQ, _EMB)
